# baseline (speedup 1.0000x reference)
import functools

import jax
import jax.numpy as jnp
from jax import lax
from jax.experimental import pallas as pl
from jax.experimental.pallas import tpu as pltpu
from jax.experimental.pallas import tpu_sc as plsc

_C = 5
_T = 512
_ROWS = 10240
_NC = 2
_NS = 16
_NW = _NC * _NS
_L = 16
_RPW = _ROWS // _NW
_GPW = _RPW // _L
_UNROLL = 4


def _sc_call(vflat, tgt):
    mesh = plsc.VectorSubcoreMesh(core_axis_name="c", subcore_axis_name="s")

    @functools.partial(
        pl.kernel, mesh=mesh,
        compiler_params=pltpu.CompilerParams(needs_layout_passes=False),
        out_type=[
            jax.ShapeDtypeStruct((_ROWS,), jnp.float32),
            jax.ShapeDtypeStruct((_NW * _L,), jnp.float32),
        ],
        scratch_types=[
            pltpu.VMEM((_L * _T,), jnp.float32),
            pltpu.VMEM((_RPW,), jnp.float32),
            pltpu.VMEM((_RPW,), jnp.float32),
            pltpu.VMEM((_L,), jnp.float32),
        ],
    )
    def _stca_sc(v_hbm, tgt_hbm, spike_hbm, lpart_hbm,
                 vbuf, tgt_buf, spike_buf, loss_buf):
        wid = lax.axis_index("s") * _NC + lax.axis_index("c")
        base_row = wid * _RPW
        pltpu.sync_copy(tgt_hbm.at[pl.ds(base_row, _RPW)], tgt_buf)

        lanes = lax.iota(jnp.int32, _L)
        zero = jnp.zeros((_L,), jnp.float32)
        one = jnp.full((_L,), 1.0, jnp.float32)
        five = jnp.full((_L,), float(_C), jnp.float32)
        big = jnp.full((_L,), 1e30, jnp.float32)
        half = jnp.full((_L,), 0.5, jnp.float32)
        base_idx = lanes * _T
        loss_acc = zero

        for g in range(_GPW):
            pltpu.sync_copy(
                v_hbm.at[pl.ds((base_row + g * _L) * _T, _L * _T)], vbuf)

            def step(_, carry):
                (idx, since, cnt, psum, pn, bc, bps, bpn, ncl, vmax) = carry
                for _u in range(_UNROLL):
                    v = plsc.load_gather(vbuf, [idx])
                    pos = v >= zero
                    poss = v > zero
                    st = pos & (since > five)
                    close = st & (cnt > zero) & (cnt < bc)
                    bc = jnp.where(close, cnt, bc)
                    bps = jnp.where(close, psum, bps)
                    bpn = jnp.where(close, pn, bpn)
                    inc_c = jnp.where(pos, one, zero)
                    sv = jnp.where(poss, v, zero)
                    inc_s = jnp.where(poss, one, zero)
                    cnt = jnp.where(st, one, cnt + inc_c)
                    psum = jnp.where(st, sv, psum + sv)
                    pn = jnp.where(st, inc_s, pn + inc_s)
                    ncl = ncl + jnp.where(st, one, zero)
                    vmax = jnp.maximum(vmax, v)
                    since = jnp.where(pos, one, since + one)
                    idx = idx + 1
                return (idx, since, cnt, psum, pn, bc, bps, bpn, ncl, vmax)

            init = (base_idx, big, zero, zero, zero, big, zero, zero, zero,
                    jnp.full((_L,), -1e30, jnp.float32))
            (_, _, cnt, psum, pn, bc, bps, bpn, ncl, vmax) = lax.fori_loop(
                0, _T // _UNROLL, step, init)

            close = (cnt > zero) & (cnt < bc)
            bps = jnp.where(close, psum, bps)
            bpn = jnp.where(close, pn, bpn)

            goff = lanes + g * _L
            tgtv = plsc.load_gather(tgt_buf, [goff])
            is_tgt = tgtv > half
            spiked = ncl > half
            contrib = jnp.where(bpn > zero,
                                bps / jnp.maximum(bpn, one), zero)
            rowloss = jnp.where(is_tgt & ~spiked, -vmax,
                                jnp.where((~is_tgt) & spiked, contrib, zero))
            loss_acc = loss_acc + rowloss
            plsc.store_scatter(spike_buf, [goff], ncl)

        loss_buf[...] = loss_acc
        pltpu.sync_copy(spike_buf, spike_hbm.at[pl.ds(base_row, _RPW)])
        pltpu.sync_copy(loss_buf, lpart_hbm.at[pl.ds(wid * _L, _L)])

    return _stca_sc(vflat, tgt)


@jax.jit
def _run(vmem, labels):
    B, N, T = vmem.shape
    tgt = (labels[:, None] == jnp.arange(N, dtype=labels.dtype)[None, :])
    spike, lpart = _sc_call(vmem.reshape(-1), tgt.reshape(-1).astype(jnp.float32))
    return jnp.sum(lpart), spike.reshape(B, N)


def kernel(vmem, vlastmem, labels):
    del vlastmem
    return _run(vmem, labels)


# speedup vs baseline: 1.1483x; 1.1483x over previous
import functools

import jax
import jax.numpy as jnp
from jax import lax
from jax.experimental import pallas as pl
from jax.experimental.pallas import tpu as pltpu
from jax.experimental.pallas import tpu_sc as plsc

_C = 5
_T = 512
_ROWS = 10240
_NC = 2
_NS = 16
_NW = _NC * _NS
_L = 16
_RPW = _ROWS // _NW
_GPW = _RPW // _L
_UNROLL = 8


def _sc_call(vflat, tgt):
    mesh = plsc.VectorSubcoreMesh(core_axis_name="c", subcore_axis_name="s")

    @functools.partial(
        pl.kernel, mesh=mesh,
        compiler_params=pltpu.CompilerParams(needs_layout_passes=False),
        out_type=[
            jax.ShapeDtypeStruct((_ROWS,), jnp.float32),
            jax.ShapeDtypeStruct((_NW * _L,), jnp.float32),
        ],
        scratch_types=[
            pltpu.VMEM((_L * _T,), jnp.float32),
            pltpu.VMEM((_L * _T,), jnp.float32),
            pltpu.VMEM((_RPW,), jnp.float32),
            pltpu.VMEM((_RPW,), jnp.float32),
            pltpu.VMEM((_L,), jnp.float32),
            pltpu.SemaphoreType.DMA,
            pltpu.SemaphoreType.DMA,
        ],
    )
    def _stca_sc(v_hbm, tgt_hbm, spike_hbm, lpart_hbm,
                 vbuf_a, vbuf_b, tgt_buf, spike_buf, loss_buf, sem_a, sem_b):
        wid = lax.axis_index("s") * _NC + lax.axis_index("c")
        base_row = wid * _RPW
        pltpu.sync_copy(tgt_hbm.at[pl.ds(base_row, _RPW)], tgt_buf)

        bufs = (vbuf_a, vbuf_b)
        sems = (sem_a, sem_b)

        def fetch(g):
            return pltpu.async_copy(
                v_hbm.at[pl.ds((base_row + g * _L) * _T, _L * _T)],
                bufs[g % 2], sems[g % 2])

        lanes = lax.iota(jnp.int32, _L)
        zero = jnp.zeros((_L,), jnp.float32)
        one = jnp.full((_L,), 1.0, jnp.float32)
        five = jnp.full((_L,), float(_C), jnp.float32)
        big = jnp.full((_L,), 1e30, jnp.float32)
        half = jnp.full((_L,), 0.5, jnp.float32)
        base_idx = lanes * _T
        loss_acc = zero

        pending = fetch(0)
        for g in range(_GPW):
            pending.wait()
            if g + 1 < _GPW:
                pending = fetch(g + 1)
            vbuf = bufs[g % 2]

            def step(_, carry, vbuf=vbuf):
                (idx, since, cnt, psum, pn, bc, bps, bpn, ncl, vmax) = carry
                for _u in range(_UNROLL):
                    v = plsc.load_gather(vbuf, [idx])
                    pos = v >= zero
                    poss = v > zero
                    st = pos & (since > five)
                    close = st & (cnt < bc)
                    bc = jnp.where(close, cnt, bc)
                    bps = jnp.where(close, psum, bps)
                    bpn = jnp.where(close, pn, bpn)
                    inc_c = jnp.where(pos, one, zero)
                    sv = jnp.where(poss, v, zero)
                    inc_s = jnp.where(poss, one, zero)
                    cnt = jnp.where(st, one, cnt + inc_c)
                    psum = jnp.where(st, sv, psum + sv)
                    pn = jnp.where(st, inc_s, pn + inc_s)
                    ncl = ncl + jnp.where(st, one, zero)
                    vmax = jnp.maximum(vmax, v)
                    since = jnp.where(pos, one, since + one)
                    idx = idx + 1
                return (idx, since, cnt, psum, pn, bc, bps, bpn, ncl, vmax)

            init = (base_idx, big, big, zero, zero, big, zero, zero, zero,
                    jnp.full((_L,), -1e30, jnp.float32))
            (_, _, cnt, psum, pn, bc, bps, bpn, ncl, vmax) = lax.fori_loop(
                0, _T // _UNROLL, step, init)

            close = cnt < bc
            bps = jnp.where(close, psum, bps)
            bpn = jnp.where(close, pn, bpn)

            goff = lanes + g * _L
            tgtv = plsc.load_gather(tgt_buf, [goff])
            is_tgt = tgtv > half
            spiked = ncl > half
            contrib = jnp.where(bpn > zero,
                                bps / jnp.maximum(bpn, one), zero)
            rowloss = jnp.where(is_tgt & ~spiked, -vmax,
                                jnp.where((~is_tgt) & spiked, contrib, zero))
            loss_acc = loss_acc + rowloss
            plsc.store_scatter(spike_buf, [goff], ncl)

        loss_buf[...] = loss_acc
        pltpu.sync_copy(spike_buf, spike_hbm.at[pl.ds(base_row, _RPW)])
        pltpu.sync_copy(loss_buf, lpart_hbm.at[pl.ds(wid * _L, _L)])

    return _stca_sc(vflat, tgt)


@jax.jit
def _run(vmem, labels):
    B, N, T = vmem.shape
    tgt = (labels[:, None] == jnp.arange(N, dtype=labels.dtype)[None, :])
    spike, lpart = _sc_call(vmem.reshape(-1), tgt.reshape(-1).astype(jnp.float32))
    return jnp.sum(lpart), spike.reshape(B, N)


def kernel(vmem, vlastmem, labels):
    del vlastmem
    return _run(vmem, labels)
